# baseline (speedup 1.0000x reference)
import functools

import jax
import jax.numpy as jnp
from jax import lax
from jax.experimental import pallas as pl
from jax.experimental.pallas import tpu as pltpu
from jax.experimental.pallas import tpu_sc as plsc

NC = 2
NS = 16
NW = NC * NS

B = 4096
S = 50
N = B * S
D = 64
POOL = 1000000

BC = 2048

CH = 400
ROWS_PER_W = N // NW
NCH = ROWS_PER_W // CH
NBUF = 4
AHEAD = 2


def _tc_transpose_kernel(t_ref, eye_ref, o_ref):
    o_ref[...] = jax.lax.dot_general(
        t_ref[...],
        eye_ref[...],
        (((0,), (0,)), ((), ())),
        preferred_element_type=jnp.float32,
        precision=jax.lax.Precision.HIGHEST,
    )


def _gather_kernel(idx_hbm, table_hbm, out_hbm, idx_v, bufs, gsem, osem):
    wid = lax.axis_index("s") * NC + lax.axis_index("c")
    base = wid * ROWS_PER_W
    pltpu.sync_copy(idx_hbm.at[wid], idx_v)

    def gdesc(t):
        return pltpu.make_async_copy(
            table_hbm.at[idx_v.at[t]], bufs.at[lax.rem(t, NBUF)], gsem
        )

    def odesc(t):
        return pltpu.make_async_copy(
            bufs.at[lax.rem(t, NBUF)], out_hbm.at[pl.ds(base + t * CH, CH)], osem
        )

    for t in range(AHEAD):
        gdesc(t).start()

    def body(t, carry):
        gdesc(t).wait()
        odesc(t).start()
        w = t - (NBUF - AHEAD)

        @pl.when(w >= 0)
        def _():
            odesc(w).wait()

        @pl.when(t + AHEAD < NCH)
        def _():
            gdesc(t + AHEAD).start()

        return carry

    lax.fori_loop(0, NCH, body, 0)

    for t in range(NCH - (NBUF - AHEAD), NCH):
        odesc(t).wait()


@jax.jit
def _run(idx_grouped, tableT):
    scratch = pl.pallas_call(
        _tc_transpose_kernel,
        grid=(pl.cdiv(POOL, BC),),
        in_specs=[
            pl.BlockSpec((64, BC), lambda i: (0, i)),
            pl.BlockSpec((64, 64), lambda i: (0, 0)),
        ],
        out_specs=pl.BlockSpec((BC, D), lambda i: (i, 0)),
        out_shape=jax.ShapeDtypeStruct((POOL, D), jnp.float32),
        compiler_params=pltpu.CompilerParams(fuse_transposed_lhs_in_matmul=True),
    )(tableT, jnp.eye(64, dtype=jnp.float32))

    gather = functools.partial(
        pl.kernel,
        out_type=jax.ShapeDtypeStruct((N, D), jnp.float32),
        mesh=plsc.VectorSubcoreMesh(core_axis_name="c", subcore_axis_name="s"),
        scratch_types=[
            pltpu.VMEM((NCH, CH), jnp.int32),
            pltpu.VMEM((NBUF, CH, D), jnp.float32),
            pltpu.SemaphoreType.DMA,
            pltpu.SemaphoreType.DMA,
        ],
        compiler_params=pltpu.CompilerParams(use_tc_tiling_on_sc=False),
    )(_gather_kernel)
    return gather(idx_grouped, scratch)


def kernel(indices, table):
    idx_grouped = indices.reshape(NW, NCH, CH).astype(jnp.int32)
    out = _run(idx_grouped, table.T)
    return out.reshape(B, S, D)


# speedup vs baseline: 1.1791x; 1.1791x over previous
import functools

import jax
import jax.numpy as jnp
from jax import lax
from jax.experimental import pallas as pl
from jax.experimental.pallas import tpu as pltpu
from jax.experimental.pallas import tpu_sc as plsc

NC = 2
NS = 16
NW = NC * NS

B = 4096
S = 50
N = B * S
D = 64
POOL = 1000000

BC = 8192

CH = 400
ROWS_PER_W = N // NW
NCH = ROWS_PER_W // CH
NBUF = 4
AHEAD = 2


def _tc_transpose_kernel(t_ref, eye_ref, o_ref):
    o_ref[...] = jax.lax.dot_general(
        t_ref[...],
        eye_ref[...],
        (((0,), (0,)), ((), ())),
        preferred_element_type=jnp.float32,
        precision=jax.lax.Precision.HIGHEST,
    )


def _gather_kernel(idx_hbm, table_hbm, out_hbm, idx_v, bufs, gsem, osem):
    wid = lax.axis_index("s") * NC + lax.axis_index("c")
    base = wid * ROWS_PER_W
    pltpu.sync_copy(idx_hbm.at[wid], idx_v)

    def gdesc(t):
        return pltpu.make_async_copy(
            table_hbm.at[idx_v.at[t]], bufs.at[lax.rem(t, NBUF)], gsem
        )

    def odesc(t):
        return pltpu.make_async_copy(
            bufs.at[lax.rem(t, NBUF)], out_hbm.at[pl.ds(base + t * CH, CH)], osem
        )

    for t in range(AHEAD):
        gdesc(t).start()

    def body(t, carry):
        gdesc(t).wait()
        odesc(t).start()
        w = t - (NBUF - AHEAD)

        @pl.when(w >= 0)
        def _():
            odesc(w).wait()

        @pl.when(t + AHEAD < NCH)
        def _():
            gdesc(t + AHEAD).start()

        return carry

    lax.fori_loop(0, NCH, body, 0)

    for t in range(NCH - (NBUF - AHEAD), NCH):
        odesc(t).wait()


@jax.jit
def _run(idx_grouped, tableT):
    scratch = pl.pallas_call(
        _tc_transpose_kernel,
        grid=(pl.cdiv(POOL, BC),),
        in_specs=[
            pl.BlockSpec((64, BC), lambda i: (0, i)),
            pl.BlockSpec((64, 64), lambda i: (0, 0)),
        ],
        out_specs=pl.BlockSpec((BC, D), lambda i: (i, 0)),
        out_shape=jax.ShapeDtypeStruct((POOL, D), jnp.float32),
        compiler_params=pltpu.CompilerParams(fuse_transposed_lhs_in_matmul=True),
    )(tableT, jnp.eye(64, dtype=jnp.float32))

    gather = functools.partial(
        pl.kernel,
        out_type=jax.ShapeDtypeStruct((N, D), jnp.float32),
        mesh=plsc.VectorSubcoreMesh(core_axis_name="c", subcore_axis_name="s"),
        scratch_types=[
            pltpu.VMEM((NCH, CH), jnp.int32),
            pltpu.VMEM((NBUF, CH, D), jnp.float32),
            pltpu.SemaphoreType.DMA,
            pltpu.SemaphoreType.DMA,
        ],
        compiler_params=pltpu.CompilerParams(use_tc_tiling_on_sc=False),
    )(_gather_kernel)
    return gather(idx_grouped, scratch)


def kernel(indices, table):
    idx_grouped = indices.reshape(NW, NCH, CH).astype(jnp.int32)
    out = _run(idx_grouped, table.T)
    return out.reshape(B, S, D)


# speedup vs baseline: 1.5237x; 1.2922x over previous
import functools

import jax
import jax.numpy as jnp
from jax import lax
from jax.experimental import pallas as pl
from jax.experimental.pallas import tpu as pltpu
from jax.experimental.pallas import tpu_sc as plsc

NC = 2
NS = 16
NW = NC * NS

B = 4096
S = 50
N = B * S
D = 64

CH = 400
ROWS_PER_W = N // NW
NCH = ROWS_PER_W // CH
NBUF = 4
AHEAD = 2


def _gather_kernel(idx_hbm, table_hbm, out_hbm, idx_v, bufs, gsem, osem):
    wid = lax.axis_index("s") * NC + lax.axis_index("c")
    base = wid * ROWS_PER_W
    pltpu.sync_copy(idx_hbm.at[wid], idx_v)

    def gdesc(t):
        return pltpu.make_async_copy(
            table_hbm.at[idx_v.at[t]], bufs.at[lax.rem(t, NBUF)], gsem
        )

    def odesc(t):
        return pltpu.make_async_copy(
            bufs.at[lax.rem(t, NBUF)], out_hbm.at[pl.ds(base + t * CH, CH)], osem
        )

    for t in range(AHEAD):
        gdesc(t).start()

    def body(t, carry):
        gdesc(t).wait()
        odesc(t).start()
        w = t - (NBUF - AHEAD)

        @pl.when(w >= 0)
        def _():
            odesc(w).wait()

        @pl.when(t + AHEAD < NCH)
        def _():
            gdesc(t + AHEAD).start()

        return carry

    lax.fori_loop(0, NCH, body, 0)

    for t in range(NCH - (NBUF - AHEAD), NCH):
        odesc(t).wait()


@jax.jit
def _run(idx_grouped, table):
    k = functools.partial(
        pl.kernel,
        out_type=jax.ShapeDtypeStruct((N, D), jnp.float32),
        mesh=plsc.VectorSubcoreMesh(core_axis_name="c", subcore_axis_name="s"),
        scratch_types=[
            pltpu.VMEM((NCH, CH), jnp.int32),
            pltpu.VMEM((NBUF, CH, D), jnp.float32),
            pltpu.SemaphoreType.DMA,
            pltpu.SemaphoreType.DMA,
        ],
        compiler_params=pltpu.CompilerParams(use_tc_tiling_on_sc=False),
    )(_gather_kernel)
    return k(idx_grouped, table)


def kernel(indices, table):
    idx_grouped = indices.reshape(NW, NCH, CH).astype(jnp.int32)
    out = _run(idx_grouped, table)
    return out.reshape(B, S, D)
